# initial kernel scaffold (unmeasured)
import jax
import jax.numpy as jnp
from jax import lax
from jax.experimental import pallas as pl
from jax.experimental.pallas import tpu as pltpu


def kernel(
    u,
):
    def body(*refs):
        pass

    out_shape = jax.ShapeDtypeStruct(..., jnp.float32)
    return pl.pallas_call(body, out_shape=out_shape)(...)


# baseline (device time: 11423 ns/iter reference)
import jax
import jax.numpy as jnp
from jax import lax
from jax.experimental import pallas as pl
from jax.experimental.pallas import tpu as pltpu

NX, NY, NZ = 2, 2, 4


def kernel(u):
    sx, sy, sz = u.shape
    gx, gy, gz = NX * sx, NY * sy, NZ * sz

    def body(u_ref, out_ref, xlo, xhi, ylo, yhi, zlo, zhi,
             sxlo, sxhi, sylo, syhi, szlo, szhi, send_sems, recv_sems):
        mx = lax.axis_index("x")
        my = lax.axis_index("y")
        mz = lax.axis_index("z")
        xn = 1 - mx
        yn = 1 - my
        zp = (mz + 1) % NZ
        zm = (mz - 1) % NZ

        barrier_sem = pltpu.get_barrier_semaphore()
        for dev in [(xn, my, mz), (mx, yn, mz), (mx, my, zp), (mx, my, zm)]:
            pl.semaphore_signal(
                barrier_sem, inc=1,
                device_id=dev, device_id_type=pl.DeviceIdType.MESH,
            )
        pl.semaphore_wait(barrier_sem, 4)

        sxlo[:, :, :] = u_ref[0:1, :, :]
        sxhi[:, :, :] = u_ref[sx - 1:sx, :, :]
        sylo[:, :, :] = u_ref[:, 0:1, :]
        syhi[:, :, :] = u_ref[:, sy - 1:sy, :]
        szlo[:, :, :] = u_ref[:, :, 0:1]
        szhi[:, :, :] = u_ref[:, :, sz - 1:sz]

        copies = [
            (sxlo, xhi, (xn, my, mz)),
            (sxhi, xlo, (xn, my, mz)),
            (sylo, yhi, (mx, yn, mz)),
            (syhi, ylo, (mx, yn, mz)),
            (szlo, zhi, (mx, my, zm)),
            (szhi, zlo, (mx, my, zp)),
        ]
        rdmas = []
        for i, (src, dst, dev) in enumerate(copies):
            rdma = pltpu.make_async_remote_copy(
                src_ref=src, dst_ref=dst,
                send_sem=send_sems.at[i], recv_sem=recv_sems.at[i],
                device_id=dev, device_id_type=pl.DeviceIdType.MESH,
            )
            rdma.start()
            rdmas.append(rdma)
        for rdma in rdmas:
            rdma.wait()

        uc = u_ref[:, :, :]
        ux = jnp.concatenate([xlo[:, :, :], uc, xhi[:, :, :]], axis=0)
        uy = jnp.concatenate([ylo[:, :, :], uc, yhi[:, :, :]], axis=1)
        uz = jnp.concatenate([zlo[:, :, :], uc, zhi[:, :, :]], axis=2)
        v = (
            ux[0:sx, :, :] + ux[2:sx + 2, :, :]
            + uy[:, 0:sy, :] + uy[:, 2:sy + 2, :]
            + uz[:, :, 0:sz] + uz[:, :, 2:sz + 2]
            - 6.0 * uc
        )

        gi = mx * sx + lax.broadcasted_iota(jnp.int32, (sx, sy, sz), 0)
        gj = my * sy + lax.broadcasted_iota(jnp.int32, (sx, sy, sz), 1)
        gk = mz * sz + lax.broadcasted_iota(jnp.int32, (sx, sy, sz), 2)
        interior = (
            (gi > 0) & (gi < gx - 1)
            & (gj > 0) & (gj < gy - 1)
            & (gk > 0) & (gk < gz - 1)
        )
        out_ref[:, :, :] = jnp.where(interior, v, 0.0)

    return pl.pallas_call(
        body,
        out_shape=jax.ShapeDtypeStruct((sx, sy, sz), u.dtype),
        in_specs=[pl.BlockSpec(memory_space=pltpu.VMEM)],
        out_specs=pl.BlockSpec(memory_space=pltpu.VMEM),
        scratch_shapes=[
            pltpu.VMEM((1, sy, sz), u.dtype),
            pltpu.VMEM((1, sy, sz), u.dtype),
            pltpu.VMEM((sx, 1, sz), u.dtype),
            pltpu.VMEM((sx, 1, sz), u.dtype),
            pltpu.VMEM((sx, sy, 1), u.dtype),
            pltpu.VMEM((sx, sy, 1), u.dtype),
            pltpu.VMEM((1, sy, sz), u.dtype),
            pltpu.VMEM((1, sy, sz), u.dtype),
            pltpu.VMEM((sx, 1, sz), u.dtype),
            pltpu.VMEM((sx, 1, sz), u.dtype),
            pltpu.VMEM((sx, sy, 1), u.dtype),
            pltpu.VMEM((sx, sy, 1), u.dtype),
            pltpu.SemaphoreType.DMA((6,)),
            pltpu.SemaphoreType.DMA((6,)),
        ],
        compiler_params=pltpu.CompilerParams(collective_id=0),
    )(u)


# device time: 7996 ns/iter; 1.4286x vs baseline; 1.4286x over previous
import jax
import jax.numpy as jnp
from jax import lax
from jax.experimental import pallas as pl
from jax.experimental.pallas import tpu as pltpu

NX, NY, NZ = 2, 2, 4


def kernel(u):
    sx, sy, sz = u.shape
    gx, gy, gz = NX * sx, NY * sy, NZ * sz

    def body(u_ref, out_ref, xlo, xhi, ylo, yhi, zlo, zhi,
             sxlo, sxhi, sylo, syhi, szlo, szhi, send_sems, recv_sems):
        mx = lax.axis_index("x")
        my = lax.axis_index("y")
        mz = lax.axis_index("z")
        xn = 1 - mx
        yn = 1 - my
        zp = mz + 1
        zm = mz - 1

        barrier_sem = pltpu.get_barrier_semaphore()
        pl.semaphore_signal(barrier_sem, inc=1, device_id=(xn, my, mz),
                            device_id_type=pl.DeviceIdType.MESH)
        pl.semaphore_signal(barrier_sem, inc=1, device_id=(mx, yn, mz),
                            device_id_type=pl.DeviceIdType.MESH)

        @pl.when(mz > 0)
        def _():
            pl.semaphore_signal(barrier_sem, inc=1, device_id=(mx, my, zm),
                                device_id_type=pl.DeviceIdType.MESH)

        @pl.when(mz < NZ - 1)
        def _():
            pl.semaphore_signal(barrier_sem, inc=1, device_id=(mx, my, zp),
                                device_id_type=pl.DeviceIdType.MESH)

        n_nbrs = 2 + jnp.where(mz > 0, 1, 0) + jnp.where(mz < NZ - 1, 1, 0)
        pl.semaphore_wait(barrier_sem, n_nbrs)

        def mk(slot, src, dst, dev):
            return pltpu.make_async_remote_copy(
                src_ref=src, dst_ref=dst,
                send_sem=send_sems.at[slot], recv_sem=recv_sems.at[slot],
                device_id=dev, device_id_type=pl.DeviceIdType.MESH,
            )

        @pl.when(mx == 1)
        def _():
            sxlo[:, :, :] = u_ref[0:1, :, :]
            mk(0, sxlo, xhi, (xn, my, mz)).start()

        @pl.when(mx == 0)
        def _():
            sxhi[:, :, :] = u_ref[sx - 1:sx, :, :]
            mk(1, sxhi, xlo, (xn, my, mz)).start()

        @pl.when(my == 1)
        def _():
            sylo[:, :, :] = u_ref[:, 0:1, :]
            mk(2, sylo, yhi, (mx, yn, mz)).start()

        @pl.when(my == 0)
        def _():
            syhi[:, :, :] = u_ref[:, sy - 1:sy, :]
            mk(3, syhi, ylo, (mx, yn, mz)).start()

        @pl.when(mz > 0)
        def _():
            szlo[:, :, :] = u_ref[:, :, 0:1]
            mk(4, szlo, zhi, (mx, my, zm)).start()

        @pl.when(mz < NZ - 1)
        def _():
            szhi[:, :, :] = u_ref[:, :, sz - 1:sz]
            mk(5, szhi, zlo, (mx, my, zp)).start()

        uc = u_ref[:, :, :]
        v_in = (
            -6.0 * uc
        )

        @pl.when(mx == 0)
        def _():
            mk(0, sxlo, xhi, (xn, my, mz)).wait_recv()

        @pl.when(mx == 1)
        def _():
            d = mk(0, sxlo, xhi, (xn, my, mz))
            d.wait_send()
            mk(1, sxhi, xlo, (xn, my, mz)).wait_recv()

        @pl.when(mx == 0)
        def _():
            mk(1, sxhi, xlo, (xn, my, mz)).wait_send()

        @pl.when(my == 0)
        def _():
            mk(2, sylo, yhi, (mx, yn, mz)).wait_recv()

        @pl.when(my == 1)
        def _():
            d = mk(2, sylo, yhi, (mx, yn, mz))
            d.wait_send()
            mk(3, syhi, ylo, (mx, yn, mz)).wait_recv()

        @pl.when(my == 0)
        def _():
            mk(3, syhi, ylo, (mx, yn, mz)).wait_send()

        @pl.when(mz > 0)
        def _():
            d = mk(4, szlo, zhi, (mx, my, zm))
            d.wait_send()
            mk(5, szhi, zlo, (mx, my, zp)).wait_recv()

        @pl.when(mz < NZ - 1)
        def _():
            d = mk(5, szhi, zlo, (mx, my, zp))
            d.wait_send()
            mk(4, szlo, zhi, (mx, my, zm)).wait_recv()

        ux = jnp.concatenate([xlo[:, :, :], uc, xhi[:, :, :]], axis=0)
        uy = jnp.concatenate([ylo[:, :, :], uc, yhi[:, :, :]], axis=1)
        uz = jnp.concatenate([zlo[:, :, :], uc, zhi[:, :, :]], axis=2)
        v = (
            v_in
            + ux[0:sx, :, :] + ux[2:sx + 2, :, :]
            + uy[:, 0:sy, :] + uy[:, 2:sy + 2, :]
            + uz[:, :, 0:sz] + uz[:, :, 2:sz + 2]
        )

        gi = mx * sx + lax.broadcasted_iota(jnp.int32, (sx, sy, sz), 0)
        gj = my * sy + lax.broadcasted_iota(jnp.int32, (sx, sy, sz), 1)
        gk = mz * sz + lax.broadcasted_iota(jnp.int32, (sx, sy, sz), 2)
        interior = (
            (gi > 0) & (gi < gx - 1)
            & (gj > 0) & (gj < gy - 1)
            & (gk > 0) & (gk < gz - 1)
        )
        out_ref[:, :, :] = jnp.where(interior, v, 0.0)

    return pl.pallas_call(
        body,
        out_shape=jax.ShapeDtypeStruct((sx, sy, sz), u.dtype),
        in_specs=[pl.BlockSpec(memory_space=pltpu.VMEM)],
        out_specs=pl.BlockSpec(memory_space=pltpu.VMEM),
        scratch_shapes=[
            pltpu.VMEM((1, sy, sz), u.dtype),
            pltpu.VMEM((1, sy, sz), u.dtype),
            pltpu.VMEM((sx, 1, sz), u.dtype),
            pltpu.VMEM((sx, 1, sz), u.dtype),
            pltpu.VMEM((sx, sy, 1), u.dtype),
            pltpu.VMEM((sx, sy, 1), u.dtype),
            pltpu.VMEM((1, sy, sz), u.dtype),
            pltpu.VMEM((1, sy, sz), u.dtype),
            pltpu.VMEM((sx, 1, sz), u.dtype),
            pltpu.VMEM((sx, 1, sz), u.dtype),
            pltpu.VMEM((sx, sy, 1), u.dtype),
            pltpu.VMEM((sx, sy, 1), u.dtype),
            pltpu.SemaphoreType.DMA((6,)),
            pltpu.SemaphoreType.DMA((6,)),
        ],
        compiler_params=pltpu.CompilerParams(collective_id=0),
    )(u)
